# Initial kernel scaffold; baseline (speedup 1.0000x reference)
import jax
import jax.numpy as jnp
from jax.experimental import pallas as pl


def kernel(tokens, precursors, aa_table, charge_table, idx_to_mass):
    raise NotImplementedError("write your pallas kernel here")


# speedup vs baseline: 4.1051x; 4.1051x over previous
import numpy as np
import jax
import jax.numpy as jnp
from jax import lax
from jax.experimental import pallas as pl
from jax.experimental.pallas import tpu as pltpu

_DIM = 512
_VOCAB = 28
_MAX_CHARGE = 10


def _terms(n):
    base = 0.001 / (2.0 * np.pi)
    scale = 10000.0 / 0.001
    return (base * scale ** (np.arange(n, dtype=np.float64) / (n - 1))).astype(np.float32)


_R64 = (1.0 / _terms(64)).astype(np.float32).reshape(1, 64)
_R128 = (1.0 / _terms(128)).astype(np.float32).reshape(1, 128)


def _embed_kernel(prec_ref, tok_ref, tok_t_ref, idx2m_ref, aa_ref, charge_ref,
                  t64_ref, t128_ref, out_ref, cum_ref):
    b = pl.program_id(0)
    mass = prec_ref[b, 0]
    cidx = prec_ref[b, 1].astype(jnp.int32) - 1

    tok = tok_ref[0]
    tok_t = tok_t_ref[0]
    idx2m = idx2m_ref[...]

    ohT = tok_t[:, :, None] == lax.broadcasted_iota(jnp.int32, (64, 16, _VOCAB), 2)
    pre_t = jnp.sum(jnp.where(ohT, idx2m[None, :, :], 0.0), axis=2)

    cum_ref[...] = pre_t

    def _scan_body(l, carry):
        cum_ref[pl.ds(l, 1), :] = cum_ref[pl.ds(l, 1), :] + cum_ref[pl.ds(l - 1, 1), :]
        return carry

    lax.fori_loop(1, 64, _scan_body, 0, unroll=True)
    suffix = (mass - cum_ref[...]).T
    pre_m = pre_t.T

    r64 = t64_ref[...][None, :, :]
    pre_arg = pre_m[:, :, None] * r64
    suf_arg = suffix[:, :, None] * r64
    pre_s, pre_c = jnp.sin(pre_arg), jnp.cos(pre_arg)
    suf_s, suf_c = jnp.sin(suf_arg), jnp.cos(suf_arg)

    oh = (tok[:, :, None] == lax.broadcasted_iota(jnp.int32, (16, 64, _VOCAB), 2))
    ohf = oh.reshape(1024, _VOCAB).astype(jnp.float32)
    tgt = lax.dot_general(ohf, aa_ref[...], (((1,), (0,)), ((), ())),
                          precision=lax.Precision.HIGHEST,
                          preferred_element_type=jnp.float32)
    tgt = tgt.reshape(16, 64, 256)

    parg = mass * t128_ref[...]
    coh = lax.broadcasted_iota(jnp.int32, (_MAX_CHARGE, 256), 0) == cidx
    crow = jnp.sum(jnp.where(coh, charge_ref[...], 0.0), axis=0, keepdims=True)
    prec256 = jnp.concatenate([jnp.sin(parg), jnp.cos(parg)], axis=1) + crow
    pas = jnp.concatenate([jnp.zeros((1, 64), jnp.float32),
                           jnp.ones((1, 64), jnp.float32)], axis=1)
    prec512 = jnp.concatenate([prec256, pas, pas], axis=1)

    out = jnp.concatenate([tgt, pre_s, pre_c, suf_s, suf_c], axis=2)
    lmask = lax.broadcasted_iota(jnp.int32, (16, 64, 1), 1) == 0
    out = out + jnp.where(lmask, prec512[None, :, :], 0.0)
    out_ref[0] = out


def kernel(tokens, precursors, aa_table, charge_table, idx_to_mass):
    B, N, L = tokens.shape
    tok_t = tokens.transpose(0, 2, 1)
    idx2m = idx_to_mass.reshape(1, _VOCAB)
    return pl.pallas_call(
        _embed_kernel,
        grid=(B,),
        in_specs=[
            pl.BlockSpec(memory_space=pltpu.SMEM),
            pl.BlockSpec((1, N, L), lambda b: (b, 0, 0)),
            pl.BlockSpec((1, L, N), lambda b: (b, 0, 0)),
            pl.BlockSpec((1, _VOCAB), lambda b: (0, 0)),
            pl.BlockSpec((_VOCAB, 256), lambda b: (0, 0)),
            pl.BlockSpec((_MAX_CHARGE, 256), lambda b: (0, 0)),
            pl.BlockSpec((1, 64), lambda b: (0, 0)),
            pl.BlockSpec((1, 128), lambda b: (0, 0)),
        ],
        out_specs=pl.BlockSpec((1, N, L, _DIM), lambda b: (b, 0, 0, 0)),
        out_shape=jax.ShapeDtypeStruct((B, N, L, _DIM), jnp.float32),
        scratch_shapes=[pltpu.VMEM((L, N), jnp.float32)],
    )(precursors, tokens, tok_t, idx2m, aa_table, charge_table,
      jnp.asarray(_R64), jnp.asarray(_R128))


# speedup vs baseline: 5.6963x; 1.3876x over previous
import numpy as np
import jax
import jax.numpy as jnp
from jax import lax
from jax.experimental import pallas as pl
from jax.experimental.pallas import tpu as pltpu

_DIM = 512
_VOCAB = 28
_MAX_CHARGE = 10


def _terms(n):
    base = 0.001 / (2.0 * np.pi)
    scale = 10000.0 / 0.001
    return base * scale ** (np.arange(n, dtype=np.float64) / (n - 1))


_R64 = (1.0 / _terms(64).astype(np.float32)).astype(np.float32).reshape(1, 64)
_R128 = (1.0 / _terms(128).astype(np.float32)).astype(np.float32).reshape(1, 128)


def _embed_kernel(prec_ref, mass_cols_ref, tok_ref, tok_lf_ref, idx2m_ref,
                  idx2m_col_ref, aa_ref, charge_ref, r64_ref, r128_ref,
                  out_ref, suf_scr, scan_scr, tab_scr):
    b = pl.program_id(0)

    @pl.when(b == 0)
    def _prologue():
        tok_lf = tok_lf_ref[...]
        ohT = tok_lf[:, :, None] == lax.broadcasted_iota(
            jnp.int32, (64, 512, _VOCAB), 2)
        pm_all = jnp.sum(jnp.where(ohT, idx2m_ref[...][None, :, :], 0.0), axis=2)
        scan_scr[...] = pm_all

        def _scan_body(l, carry):
            scan_scr[pl.ds(l, 1), :] = (scan_scr[pl.ds(l, 1), :]
                                        + scan_scr[pl.ds(l - 1, 1), :])
            return carry

        lax.fori_loop(1, 64, _scan_body, 0, unroll=True)
        suf_scr[...] = (mass_cols_ref[...] - scan_scr[...]).T

        tab_scr[:, 0:256] = aa_ref[...]
        parg = idx2m_col_ref[...] * r64_ref[...]
        tab_scr[:, 256:320] = jnp.sin(parg)
        tab_scr[:, 320:384] = jnp.cos(parg)

    mass = prec_ref[b, 0]
    cidx = prec_ref[b, 1].astype(jnp.int32) - 1

    tok = tok_ref[0]
    oh = tok[:, :, None] == lax.broadcasted_iota(jnp.int32, (16, 64, _VOCAB), 2)
    ohf = oh.reshape(1024, _VOCAB).astype(jnp.float32)
    g = lax.dot_general(ohf, tab_scr[...], (((1,), (0,)), ((), ())),
                        precision=lax.Precision.HIGHEST,
                        preferred_element_type=jnp.float32)
    g = g.reshape(16, 64, 384)

    suffix = suf_scr[pl.ds(b * 16, 16), :]
    suf_arg = suffix[:, :, None] * r64_ref[...][None, :, :]
    suf_s, suf_c = jnp.sin(suf_arg), jnp.cos(suf_arg)

    parg = mass * r128_ref[...]
    coh = lax.broadcasted_iota(jnp.int32, (_MAX_CHARGE, 256), 0) == cidx
    crow = jnp.sum(jnp.where(coh, charge_ref[...], 0.0), axis=0, keepdims=True)
    prec256 = jnp.concatenate([jnp.sin(parg), jnp.cos(parg)], axis=1) + crow
    pas = jnp.concatenate([jnp.zeros((1, 64), jnp.float32),
                           jnp.ones((1, 64), jnp.float32)], axis=1)
    prec512 = jnp.concatenate([prec256, pas, pas], axis=1)

    out = jnp.concatenate([g, suf_s, suf_c], axis=2)
    lmask = lax.broadcasted_iota(jnp.int32, (16, 64, 1), 1) == 0
    out = out + jnp.where(lmask, prec512[None, :, :], 0.0)
    out_ref[0] = out


def kernel(tokens, precursors, aa_table, charge_table, idx_to_mass):
    B, N, L = tokens.shape
    tok_lf = tokens.transpose(2, 0, 1).reshape(L, B * N)
    mass_cols = jnp.repeat(precursors[:, 0], N).reshape(1, B * N)
    idx2m = idx_to_mass.reshape(1, _VOCAB)
    idx2m_col = idx_to_mass.reshape(_VOCAB, 1)
    return pl.pallas_call(
        _embed_kernel,
        grid=(B,),
        in_specs=[
            pl.BlockSpec(memory_space=pltpu.SMEM),
            pl.BlockSpec((1, B * N), lambda b: (0, 0)),
            pl.BlockSpec((1, N, L), lambda b: (b, 0, 0)),
            pl.BlockSpec((L, B * N), lambda b: (0, 0)),
            pl.BlockSpec((1, _VOCAB), lambda b: (0, 0)),
            pl.BlockSpec((_VOCAB, 1), lambda b: (0, 0)),
            pl.BlockSpec((_VOCAB, 256), lambda b: (0, 0)),
            pl.BlockSpec((_MAX_CHARGE, 256), lambda b: (0, 0)),
            pl.BlockSpec((1, 64), lambda b: (0, 0)),
            pl.BlockSpec((1, 128), lambda b: (0, 0)),
        ],
        out_specs=pl.BlockSpec((1, N, L, _DIM), lambda b: (b, 0, 0, 0)),
        out_shape=jax.ShapeDtypeStruct((B, N, L, _DIM), jnp.float32),
        scratch_shapes=[pltpu.VMEM((B * N, L), jnp.float32),
                        pltpu.VMEM((L, B * N), jnp.float32),
                        pltpu.VMEM((_VOCAB, 384), jnp.float32)],
    )(precursors, mass_cols, tokens, tok_lf, idx2m, idx2m_col,
      aa_table, charge_table, jnp.asarray(_R64), jnp.asarray(_R128))


# speedup vs baseline: 6.1903x; 1.0867x over previous
import numpy as np
import jax
import jax.numpy as jnp
from jax import lax
from jax.experimental import pallas as pl
from jax.experimental.pallas import tpu as pltpu

_DIM = 512
_VOCAB = 28
_MAX_CHARGE = 10


def _terms(n):
    base = 0.001 / (2.0 * np.pi)
    scale = 10000.0 / 0.001
    return base * scale ** (np.arange(n, dtype=np.float64) / (n - 1))


_R64 = (1.0 / _terms(64).astype(np.float32)).astype(np.float32).reshape(1, 64)
_R128 = (1.0 / _terms(128).astype(np.float32)).astype(np.float32).reshape(1, 128)


def _embed_kernel(prec_ref, mass_cols_ref, mass_col_ref, charge_col_ref,
                  tok_ref, tok_lf_ref, idx2m_s_ref, idx2m_col_ref, aa_ref,
                  charge_ref, r64_ref, r128_ref,
                  out_ref, suf_scr, scan_scr, tab_scr, prow_scr):
    b = pl.program_id(0)

    @pl.when(b == 0)
    def _prologue():
        tok_lf = tok_lf_ref[...]
        pm_all = jnp.zeros((64, 512), jnp.float32)
        for v in range(_VOCAB):
            pm_all = jnp.where(tok_lf == v, idx2m_s_ref[v], pm_all)
        scan_scr[...] = pm_all

        def _scan_body(l, carry):
            scan_scr[pl.ds(l, 1), :] = (scan_scr[pl.ds(l, 1), :]
                                        + scan_scr[pl.ds(l - 1, 1), :])
            return carry

        lax.fori_loop(1, 64, _scan_body, 0, unroll=True)
        suf_scr[...] = (mass_cols_ref[...] - scan_scr[...]).T

        tab_scr[:, 0:256] = aa_ref[...]
        targ = idx2m_col_ref[...] * r64_ref[...]
        tab_scr[:, 256:320] = jnp.sin(targ)
        tab_scr[:, 320:384] = jnp.cos(targ)

        marg = mass_col_ref[...] * r128_ref[...]
        cidx = charge_col_ref[...].astype(jnp.int32) - 1
        coh = (lax.broadcasted_iota(jnp.int32, (32, _MAX_CHARGE), 1)
               == cidx).astype(jnp.float32)
        crow = lax.dot_general(coh, charge_ref[...], (((1,), (0,)), ((), ())),
                               precision=lax.Precision.HIGHEST,
                               preferred_element_type=jnp.float32)
        prow_scr[:, 0:256] = jnp.concatenate(
            [jnp.sin(marg), jnp.cos(marg)], axis=1) + crow
        lane = lax.broadcasted_iota(jnp.int32, (32, 256), 1)
        prow_scr[:, 256:512] = jnp.where((lane & 64) != 0, 1.0, 0.0)

    tok = tok_ref[0]
    oh = tok[:, :, None] == lax.broadcasted_iota(jnp.int32, (16, 64, _VOCAB), 2)
    ohf = oh.reshape(1024, _VOCAB).astype(jnp.float32)
    g = lax.dot_general(ohf, tab_scr[...], (((1,), (0,)), ((), ())),
                        precision=lax.Precision.HIGHEST,
                        preferred_element_type=jnp.float32)
    g = g.reshape(16, 64, 384)

    suffix = suf_scr[pl.ds(b * 16, 16), :]
    suf_arg = suffix[:, :, None] * r64_ref[...][None, :, :]
    suf_s, suf_c = jnp.sin(suf_arg), jnp.cos(suf_arg)

    out_ref[0] = jnp.concatenate([g, suf_s, suf_c], axis=2)
    prec512 = prow_scr[pl.ds(b, 1), :]
    out_ref[0, :, 0, :] = out_ref[0, :, 0, :] + prec512


def kernel(tokens, precursors, aa_table, charge_table, idx_to_mass):
    B, N, L = tokens.shape
    tok_lf = tokens.transpose(2, 0, 1).reshape(L, B * N)
    mass_cols = jnp.repeat(precursors[:, 0], N).reshape(1, B * N)
    mass_col = precursors[:, 0].reshape(B, 1)
    charge_col = precursors[:, 1].reshape(B, 1)
    idx2m_col = idx_to_mass.reshape(_VOCAB, 1)
    return pl.pallas_call(
        _embed_kernel,
        grid=(B,),
        in_specs=[
            pl.BlockSpec(memory_space=pltpu.SMEM),
            pl.BlockSpec((1, B * N), lambda b: (0, 0)),
            pl.BlockSpec((B, 1), lambda b: (0, 0)),
            pl.BlockSpec((B, 1), lambda b: (0, 0)),
            pl.BlockSpec((1, N, L), lambda b: (b, 0, 0)),
            pl.BlockSpec((L, B * N), lambda b: (0, 0)),
            pl.BlockSpec(memory_space=pltpu.SMEM),
            pl.BlockSpec((_VOCAB, 1), lambda b: (0, 0)),
            pl.BlockSpec((_VOCAB, 256), lambda b: (0, 0)),
            pl.BlockSpec((_MAX_CHARGE, 256), lambda b: (0, 0)),
            pl.BlockSpec((1, 64), lambda b: (0, 0)),
            pl.BlockSpec((1, 128), lambda b: (0, 0)),
        ],
        out_specs=pl.BlockSpec((1, N, L, _DIM), lambda b: (b, 0, 0, 0)),
        out_shape=jax.ShapeDtypeStruct((B, N, L, _DIM), jnp.float32),
        scratch_shapes=[pltpu.VMEM((B * N, L), jnp.float32),
                        pltpu.VMEM((L, B * N), jnp.float32),
                        pltpu.VMEM((_VOCAB, 384), jnp.float32),
                        pltpu.VMEM((B, _DIM), jnp.float32)],
    )(precursors, mass_cols, mass_col, charge_col, tokens, tok_lf,
      idx_to_mass, idx2m_col, aa_table, charge_table,
      jnp.asarray(_R64), jnp.asarray(_R128))


# speedup vs baseline: 7.6314x; 1.2328x over previous
import numpy as np
import jax
import jax.numpy as jnp
from jax import lax
from jax.experimental import pallas as pl
from jax.experimental.pallas import tpu as pltpu

_DIM = 512
_VOCAB = 28
_MAX_CHARGE = 10


def _terms(n):
    base = 0.001 / (2.0 * np.pi)
    scale = 10000.0 / 0.001
    return base * scale ** (np.arange(n, dtype=np.float64) / (n - 1))


_R64 = (1.0 / _terms(64).astype(np.float32)).astype(np.float32).reshape(1, 64)
_R128 = (1.0 / _terms(128).astype(np.float32)).astype(np.float32).reshape(1, 128)


def _embed_kernel(prec_ref, mass_cols_ref, mass_col_ref, charge_col_ref,
                  tok_ref, tok_lf_ref, idx2m_s_ref, idx2m_col_ref, aa_ref,
                  charge_ref, r64_ref, r128_ref,
                  out_ref, suf_scr, scan_scr, tab_scr, prow_scr):
    b = pl.program_id(0)

    @pl.when(b == 0)
    def _prologue():
        tok_lf = tok_lf_ref[...]
        pm_all = jnp.zeros((64, 512), jnp.float32)
        for v in range(_VOCAB):
            pm_all = jnp.where(tok_lf == v, idx2m_s_ref[v], pm_all)
        scan_scr[...] = pm_all

        def _scan_body(l, carry):
            scan_scr[pl.ds(l, 1), :] = (scan_scr[pl.ds(l, 1), :]
                                        + scan_scr[pl.ds(l - 1, 1), :])
            return carry

        lax.fori_loop(1, 64, _scan_body, 0, unroll=True)
        suf_all = mass_cols_ref[...] - scan_scr[...]
        suf_eo = suf_all.reshape(32, 2, 512).transpose(1, 0, 2).reshape(64, 512)
        suf_scr[...] = suf_eo.T

        tab_scr[:, 0:256] = aa_ref[...]
        targ = idx2m_col_ref[...] * r64_ref[...]
        tab_scr[:, 256:320] = jnp.sin(targ)
        tab_scr[:, 320:384] = jnp.cos(targ)

        marg = mass_col_ref[...] * r128_ref[...]
        cidx = charge_col_ref[...].astype(jnp.int32) - 1
        coh = (lax.broadcasted_iota(jnp.int32, (32, _MAX_CHARGE), 1)
               == cidx).astype(jnp.float32)
        crow = lax.dot_general(coh, charge_ref[...], (((1,), (0,)), ((), ())),
                               precision=lax.Precision.HIGHEST,
                               preferred_element_type=jnp.float32)
        prow_scr[:, 0:256] = jnp.concatenate(
            [jnp.sin(marg), jnp.cos(marg)], axis=1) + crow
        lane = lax.broadcasted_iota(jnp.int32, (32, 256), 1)
        prow_scr[:, 256:512] = jnp.where((lane & 64) != 0, 1.0, 0.0)

    tok = tok_ref[0]
    oh = tok[:, :, None] == lax.broadcasted_iota(jnp.int32, (16, 64, _VOCAB), 2)
    ohf = oh.reshape(1024, _VOCAB).astype(jnp.float32)
    g = lax.dot_general(ohf, tab_scr[...], (((1,), (0,)), ((), ())),
                        precision=lax.Precision.HIGHEST,
                        preferred_element_type=jnp.float32)
    g = g.reshape(16, 64, 384)

    sfp = suf_scr[pl.ds(b * 16, 16), :]
    r64 = r64_ref[...][None, :, :]
    arg_p = jnp.concatenate([sfp[:, 0:32, None] * r64,
                             sfp[:, 32:64, None] * r64], axis=2)
    sin_p, cos_p = jnp.sin(arg_p), jnp.cos(arg_p)
    enc_e = jnp.concatenate([sin_p[:, :, 0:64], cos_p[:, :, 0:64]], axis=2)
    enc_o = jnp.concatenate([sin_p[:, :, 64:128], cos_p[:, :, 64:128]], axis=2)
    enc = jnp.stack([enc_e, enc_o], axis=2).reshape(16, 64, 128)

    out_ref[0] = jnp.concatenate([g, enc], axis=2)
    prec512 = prow_scr[pl.ds(b, 1), :]
    out_ref[0, :, 0, :] = out_ref[0, :, 0, :] + prec512


def kernel(tokens, precursors, aa_table, charge_table, idx_to_mass):
    B, N, L = tokens.shape
    tok_lf = tokens.transpose(2, 0, 1).reshape(L, B * N)
    mass_cols = jnp.repeat(precursors[:, 0], N).reshape(1, B * N)
    mass_col = precursors[:, 0].reshape(B, 1)
    charge_col = precursors[:, 1].reshape(B, 1)
    idx2m_col = idx_to_mass.reshape(_VOCAB, 1)
    return pl.pallas_call(
        _embed_kernel,
        grid=(B,),
        in_specs=[
            pl.BlockSpec(memory_space=pltpu.SMEM),
            pl.BlockSpec((1, B * N), lambda b: (0, 0)),
            pl.BlockSpec((B, 1), lambda b: (0, 0)),
            pl.BlockSpec((B, 1), lambda b: (0, 0)),
            pl.BlockSpec((1, N, L), lambda b: (b, 0, 0)),
            pl.BlockSpec((L, B * N), lambda b: (0, 0)),
            pl.BlockSpec(memory_space=pltpu.SMEM),
            pl.BlockSpec((_VOCAB, 1), lambda b: (0, 0)),
            pl.BlockSpec((_VOCAB, 256), lambda b: (0, 0)),
            pl.BlockSpec((_MAX_CHARGE, 256), lambda b: (0, 0)),
            pl.BlockSpec((1, 64), lambda b: (0, 0)),
            pl.BlockSpec((1, 128), lambda b: (0, 0)),
        ],
        out_specs=pl.BlockSpec((1, N, L, _DIM), lambda b: (b, 0, 0, 0)),
        out_shape=jax.ShapeDtypeStruct((B, N, L, _DIM), jnp.float32),
        scratch_shapes=[pltpu.VMEM((B * N, L), jnp.float32),
                        pltpu.VMEM((L, B * N), jnp.float32),
                        pltpu.VMEM((_VOCAB, 384), jnp.float32),
                        pltpu.VMEM((B, _DIM), jnp.float32)],
    )(precursors, mass_cols, mass_col, charge_col, tokens, tok_lf,
      idx_to_mass, idx2m_col, aa_table, charge_table,
      jnp.asarray(_R64), jnp.asarray(_R128))
